# baseline (device time: 16844 ns/iter reference)
import jax
import jax.numpy as jnp
from jax import lax
from jax.experimental import pallas as pl
from jax.experimental.pallas import tpu as pltpu

C = 8


def kernel(x):
    m, n = x.shape
    h = m // 2
    ch = h // C

    def body(x_hbm, out_hbm, xl_ref, sx_ref, rx_ref, red_ref, ry_ref,
             in_sems, out_sems, sx_sems, rx_sems, sy_sems, ry_sems):
        my_x = lax.axis_index("x")
        my_y = lax.axis_index("y")
        xn = (1 - my_x, my_y)
        yn = (my_x, 1 - my_y)
        base = my_y * h
        obase = (1 - my_y) * h

        in_cpy = []
        for c in range(C):
            s = pl.ds(c * ch, ch)
            cpy = pltpu.make_async_copy(
                x_hbm.at[pl.ds(base + c * ch, ch), :], xl_ref.at[s],
                in_sems.at[c],
            )
            cpy.start()
            in_cpy.append(cpy)

        barrier = pltpu.get_barrier_semaphore()
        for nbr in (xn, yn):
            pl.semaphore_signal(
                barrier, inc=1, device_id=nbr,
                device_id_type=pl.DeviceIdType.MESH,
            )
        pl.semaphore_wait(barrier, 2)

        rdma_x = []
        rdma_y = []
        for c in range(C):
            s = pl.ds(c * ch, ch)
            rdma_x.append(pltpu.make_async_remote_copy(
                src_ref=sx_ref.at[s],
                dst_ref=rx_ref.at[s],
                send_sem=sx_sems.at[c],
                recv_sem=rx_sems.at[c],
                device_id=xn,
                device_id_type=pl.DeviceIdType.MESH,
            ))
            rdma_y.append(pltpu.make_async_remote_copy(
                src_ref=red_ref.at[s],
                dst_ref=ry_ref.at[s],
                send_sem=sy_sems.at[c],
                recv_sem=ry_sems.at[c],
                device_id=yn,
                device_id_type=pl.DeviceIdType.MESH,
            ))

        for c in range(C):
            s = pl.ds(c * ch, ch)
            in_cpy[c].wait()
            sx_ref[s, :] = xl_ref[s, :].astype(jnp.bfloat16)
            rdma_x[c].start()

        out_cpy = []
        for c in range(C):
            s = pl.ds(c * ch, ch)
            rdma_x[c].wait_recv()
            red_ref[s, :] = sx_ref[s, :] + rx_ref[s, :]
            rdma_y[c].start()
            ocp = pltpu.make_async_copy(
                red_ref.at[s], out_hbm.at[pl.ds(base + c * ch, ch), :],
                out_sems.at[c],
            )
            ocp.start()
            out_cpy.append(ocp)

        for c in range(C):
            s = pl.ds(c * ch, ch)
            rdma_y[c].wait_recv()
            ocp = pltpu.make_async_copy(
                ry_ref.at[s], out_hbm.at[pl.ds(obase + c * ch, ch), :],
                out_sems.at[C + c],
            )
            ocp.start()
            out_cpy.append(ocp)

        for cpy in out_cpy:
            cpy.wait()
        for c in range(C):
            rdma_x[c].wait_send()
            rdma_y[c].wait_send()

    return pl.pallas_call(
        body,
        out_shape=jax.ShapeDtypeStruct((m, n), jnp.bfloat16),
        in_specs=[pl.BlockSpec(memory_space=pl.ANY)],
        out_specs=pl.BlockSpec(memory_space=pl.ANY),
        scratch_shapes=[
            pltpu.VMEM((h, n), jnp.float32),
            pltpu.VMEM((h, n), jnp.bfloat16),
            pltpu.VMEM((h, n), jnp.bfloat16),
            pltpu.VMEM((h, n), jnp.bfloat16),
            pltpu.VMEM((h, n), jnp.bfloat16),
            pltpu.SemaphoreType.DMA((C,)),
            pltpu.SemaphoreType.DMA((2 * C,)),
            pltpu.SemaphoreType.DMA((C,)),
            pltpu.SemaphoreType.DMA((C,)),
            pltpu.SemaphoreType.DMA((C,)),
            pltpu.SemaphoreType.DMA((C,)),
        ],
        compiler_params=pltpu.CompilerParams(collective_id=0),
    )(x)
